# baseline (speedup 1.0000x reference)
import functools

import jax
import jax.numpy as jnp
from jax import lax
from jax.experimental import pallas as pl
from jax.experimental.pallas import tpu as pltpu
from jax.experimental.pallas import tpu_sc as plsc

_V = 1000000
_D = 64
_DP = 128
_BATCH = 4096
_HIST = 200
_B = _BATCH * _HIST
_NC = 2
_NS = 16
_NW = _NC * _NS
_BPW = _BATCH // _NW
_IPW = _BPW * _HIST
_C0 = 128
_C1 = _HIST - _C0
_TBLK = 512


def _tpose_body(t_ref, o_ref):
    x = jnp.concatenate(
        [t_ref[...], jnp.zeros((_DP - _D, _TBLK), jnp.float32)], axis=0)
    o_ref[...] = x.T


_tpose = pl.pallas_call(
    _tpose_body,
    grid=(pl.cdiv(_V, _TBLK),),
    in_specs=[pl.BlockSpec((_D, _TBLK), lambda i: (0, i))],
    out_specs=pl.BlockSpec((_TBLK, _DP), lambda i: (i, 0)),
    out_shape=jax.ShapeDtypeStruct((_V, _DP), jnp.float32),
)

_mesh = plsc.VectorSubcoreMesh(core_axis_name="c", subcore_axis_name="s")


@functools.partial(
    pl.kernel,
    mesh=_mesh,
    out_type=jax.ShapeDtypeStruct((_BATCH, _HIST, _DP), jnp.float32),
    scratch_types=[
        pltpu.VMEM((_IPW,), jnp.int32),
        pltpu.VMEM((_HIST, _DP), jnp.float32),
        pltpu.VMEM((_HIST, _DP), jnp.float32),
        pltpu.SemaphoreType.DMA,
        pltpu.SemaphoreType.DMA,
        pltpu.SemaphoreType.DMA,
        pltpu.SemaphoreType.DMA,
    ],
)
def _gather(idx_hbm, table_hbm, out_hbm, idx_v, rows0, rows1, g0, g1, s0, s1):
    wid = lax.axis_index("s") * _NC + lax.axis_index("c")
    b0 = wid * _BPW

    pltpu.sync_copy(idx_hbm.at[pl.ds(wid * _IPW, _IPW)], idx_v)

    def fire_gather(j, rows, gsem):
        pltpu.async_copy(
            table_hbm.at[idx_v.at[pl.ds(j * _HIST, _C0)]],
            rows.at[pl.ds(0, _C0)],
            gsem,
        )
        pltpu.async_copy(
            table_hbm.at[idx_v.at[pl.ds(j * _HIST + _C0, _C1)]],
            rows.at[pl.ds(_C0, _C1)],
            gsem,
        )

    def drain_gather(rows, gsem):
        pltpu.make_async_copy(table_hbm.at[idx_v.at[pl.ds(0, _C0)]],
                              rows.at[pl.ds(0, _C0)], gsem).wait()
        pltpu.make_async_copy(table_hbm.at[idx_v.at[pl.ds(0, _C1)]],
                              rows.at[pl.ds(_C0, _C1)], gsem).wait()

    def fire_store(j, rows, ssem):
        pltpu.async_copy(rows, out_hbm.at[b0 + j], ssem)

    def wait_store(j, rows, ssem):
        pltpu.make_async_copy(rows, out_hbm.at[b0 + j], ssem).wait()

    fire_gather(0, rows0, g0)

    def body(jj, carry):
        j = jj * 2
        drain_gather(rows0, g0)

        @pl.when(j > 0)
        def _():
            wait_store(j - 1, rows1, s1)

        fire_gather(j + 1, rows1, g1)
        fire_store(j, rows0, s0)

        drain_gather(rows1, g1)
        wait_store(j, rows0, s0)

        @pl.when(j + 2 < _BPW)
        def _():
            fire_gather(j + 2, rows0, g0)

        fire_store(j + 1, rows1, s1)
        return carry

    lax.fori_loop(0, _BPW // 2, body, 0)
    wait_store(_BPW - 1, rows1, s1)


def kernel(traj, table):
    idx = traj.reshape(-1).astype(jnp.int32)
    table_rows = _tpose(table.T)
    out_p = _gather(idx, table_rows)
    return out_p[:, :, :_D]


# speedup vs baseline: 1.7649x; 1.7649x over previous
import functools

import jax
import jax.numpy as jnp
from jax import lax
from jax.experimental import pallas as pl
from jax.experimental.pallas import tpu as pltpu
from jax.experimental.pallas import tpu_sc as plsc

_V = 1000000
_D = 64
_DP = 128
_BATCH = 4096
_HIST = 200
_B = _BATCH * _HIST
_NC = 2
_NS = 16
_NW = _NC * _NS
_BPW = _BATCH // _NW
_IPW = _BPW * _HIST
_C0 = 128
_C1 = _HIST - _C0
_TBLK = 2048


def _tpose_body(t_ref, o_ref):
    x = jnp.concatenate(
        [t_ref[...], jnp.zeros((_DP - _D, _TBLK), jnp.float32)], axis=0)
    o_ref[...] = x.T


_tpose = pl.pallas_call(
    _tpose_body,
    grid=(pl.cdiv(_V, _TBLK),),
    in_specs=[pl.BlockSpec((_D, _TBLK), lambda i: (0, i))],
    out_specs=pl.BlockSpec((_TBLK, _DP), lambda i: (i, 0)),
    out_shape=jax.ShapeDtypeStruct((_V, _DP), jnp.float32),
)

_mesh = plsc.VectorSubcoreMesh(core_axis_name="c", subcore_axis_name="s")


@functools.partial(
    pl.kernel,
    mesh=_mesh,
    out_type=jax.ShapeDtypeStruct((_BATCH, _HIST, _DP), jnp.float32),
    scratch_types=[
        pltpu.VMEM((_IPW,), jnp.int32),
        pltpu.VMEM((_HIST, _DP), jnp.float32),
        pltpu.VMEM((_HIST, _DP), jnp.float32),
        pltpu.SemaphoreType.DMA,
        pltpu.SemaphoreType.DMA,
        pltpu.SemaphoreType.DMA,
        pltpu.SemaphoreType.DMA,
    ],
)
def _gather(idx_hbm, table_hbm, out_hbm, idx_v, rows0, rows1, g0, g1, s0, s1):
    wid = lax.axis_index("s") * _NC + lax.axis_index("c")
    b0 = wid * _BPW

    pltpu.sync_copy(idx_hbm.at[pl.ds(wid * _IPW, _IPW)], idx_v)

    def fire_gather(j, rows, gsem):
        pltpu.async_copy(
            table_hbm.at[idx_v.at[pl.ds(j * _HIST, _C0)]],
            rows.at[pl.ds(0, _C0)],
            gsem,
        )
        pltpu.async_copy(
            table_hbm.at[idx_v.at[pl.ds(j * _HIST + _C0, _C1)]],
            rows.at[pl.ds(_C0, _C1)],
            gsem,
        )

    def drain_gather(rows, gsem):
        pltpu.make_async_copy(table_hbm.at[idx_v.at[pl.ds(0, _C0)]],
                              rows.at[pl.ds(0, _C0)], gsem).wait()
        pltpu.make_async_copy(table_hbm.at[idx_v.at[pl.ds(0, _C1)]],
                              rows.at[pl.ds(_C0, _C1)], gsem).wait()

    def fire_store(j, rows, ssem):
        pltpu.async_copy(rows, out_hbm.at[b0 + j], ssem)

    def wait_store(j, rows, ssem):
        pltpu.make_async_copy(rows, out_hbm.at[b0 + j], ssem).wait()

    fire_gather(0, rows0, g0)

    def body(jj, carry):
        j = jj * 2
        drain_gather(rows0, g0)

        @pl.when(j > 0)
        def _():
            wait_store(j - 1, rows1, s1)

        fire_gather(j + 1, rows1, g1)
        fire_store(j, rows0, s0)

        drain_gather(rows1, g1)
        wait_store(j, rows0, s0)

        @pl.when(j + 2 < _BPW)
        def _():
            fire_gather(j + 2, rows0, g0)

        fire_store(j + 1, rows1, s1)
        return carry

    lax.fori_loop(0, _BPW // 2, body, 0)
    wait_store(_BPW - 1, rows1, s1)


def kernel(traj, table):
    idx = traj.reshape(-1).astype(jnp.int32)
    table_rows = _tpose(table.T)
    out_p = _gather(idx, table_rows)
    return out_p[:, :, :_D]


# speedup vs baseline: 2.2188x; 1.2572x over previous
import functools

import jax
import jax.numpy as jnp
from jax import lax
from jax.experimental import pallas as pl
from jax.experimental.pallas import tpu as pltpu
from jax.experimental.pallas import tpu_sc as plsc

_V = 1000000
_D = 64
_DP = 128
_BATCH = 4096
_HIST = 200
_B = _BATCH * _HIST
_NC = 2
_NS = 16
_NW = _NC * _NS
_BPW = _BATCH // _NW
_IPW = _BPW * _HIST
_C0 = 128
_C1 = _HIST - _C0
_TBLK = 8192


def _tpose_body(t_ref, o_ref):
    x = jnp.concatenate(
        [t_ref[...], jnp.zeros((_DP - _D, _TBLK), jnp.float32)], axis=0)
    o_ref[...] = x.T


_tpose = pl.pallas_call(
    _tpose_body,
    grid=(pl.cdiv(_V, _TBLK),),
    in_specs=[pl.BlockSpec((_D, _TBLK), lambda i: (0, i))],
    out_specs=pl.BlockSpec((_TBLK, _DP), lambda i: (i, 0)),
    out_shape=jax.ShapeDtypeStruct((_V, _DP), jnp.float32),
)

_mesh = plsc.VectorSubcoreMesh(core_axis_name="c", subcore_axis_name="s")


@functools.partial(
    pl.kernel,
    mesh=_mesh,
    out_type=jax.ShapeDtypeStruct((_BATCH, _HIST, _DP), jnp.float32),
    scratch_types=[
        pltpu.VMEM((_IPW,), jnp.int32),
        pltpu.VMEM((_HIST, _DP), jnp.float32),
        pltpu.VMEM((_HIST, _DP), jnp.float32),
        pltpu.SemaphoreType.DMA,
        pltpu.SemaphoreType.DMA,
        pltpu.SemaphoreType.DMA,
        pltpu.SemaphoreType.DMA,
    ],
)
def _gather(idx_hbm, table_hbm, out_hbm, idx_v, rows0, rows1, g0, g1, s0, s1):
    wid = lax.axis_index("s") * _NC + lax.axis_index("c")
    b0 = wid * _BPW

    pltpu.sync_copy(idx_hbm.at[pl.ds(wid * _IPW, _IPW)], idx_v)

    def fire_gather(j, rows, gsem):
        pltpu.async_copy(
            table_hbm.at[idx_v.at[pl.ds(j * _HIST, _C0)]],
            rows.at[pl.ds(0, _C0)],
            gsem,
        )
        pltpu.async_copy(
            table_hbm.at[idx_v.at[pl.ds(j * _HIST + _C0, _C1)]],
            rows.at[pl.ds(_C0, _C1)],
            gsem,
        )

    def drain_gather(rows, gsem):
        pltpu.make_async_copy(table_hbm.at[idx_v.at[pl.ds(0, _C0)]],
                              rows.at[pl.ds(0, _C0)], gsem).wait()
        pltpu.make_async_copy(table_hbm.at[idx_v.at[pl.ds(0, _C1)]],
                              rows.at[pl.ds(_C0, _C1)], gsem).wait()

    def fire_store(j, rows, ssem):
        pltpu.async_copy(rows, out_hbm.at[b0 + j], ssem)

    def wait_store(j, rows, ssem):
        pltpu.make_async_copy(rows, out_hbm.at[b0 + j], ssem).wait()

    fire_gather(0, rows0, g0)

    def body(jj, carry):
        j = jj * 2
        drain_gather(rows0, g0)

        @pl.when(j > 0)
        def _():
            wait_store(j - 1, rows1, s1)

        fire_gather(j + 1, rows1, g1)
        fire_store(j, rows0, s0)

        drain_gather(rows1, g1)
        wait_store(j, rows0, s0)

        @pl.when(j + 2 < _BPW)
        def _():
            fire_gather(j + 2, rows0, g0)

        fire_store(j + 1, rows1, s1)
        return carry

    lax.fori_loop(0, _BPW // 2, body, 0)
    wait_store(_BPW - 1, rows1, s1)


def kernel(traj, table):
    idx = traj.reshape(-1).astype(jnp.int32)
    table_rows = _tpose(table.T)
    out_p = _gather(idx, table_rows)
    return out_p[:, :, :_D]


# speedup vs baseline: 2.6030x; 1.1731x over previous
import functools

import jax
import jax.numpy as jnp
from jax import lax
from jax.experimental import pallas as pl
from jax.experimental.pallas import tpu as pltpu
from jax.experimental.pallas import tpu_sc as plsc

_V = 1000000
_D = 64
_DP = 128
_BATCH = 4096
_HIST = 200
_B = _BATCH * _HIST
_NC = 2
_NS = 16
_NW = _NC * _NS
_BPW = _BATCH // _NW
_IPW = _BPW * _HIST
_C0 = 128
_C1 = _HIST - _C0
_TBLK = 8192


def _tpose_body(t_ref, o_ref):
    x = jnp.concatenate(
        [t_ref[...], jnp.zeros((_DP - _D, _TBLK), jnp.float32)], axis=0)
    o_ref[...] = x.T


_tpose = pl.pallas_call(
    _tpose_body,
    grid=(pl.cdiv(_V, _TBLK),),
    in_specs=[pl.BlockSpec((_D, _TBLK), lambda i: (0, i))],
    out_specs=pl.BlockSpec((_TBLK, _DP), lambda i: (i, 0)),
    out_shape=jax.ShapeDtypeStruct((_V, _DP), jnp.float32),
)

_mesh = plsc.VectorSubcoreMesh(core_axis_name="c", subcore_axis_name="s")


@functools.partial(
    pl.kernel,
    mesh=_mesh,
    out_type=jax.ShapeDtypeStruct((_BATCH, _HIST, _DP), jnp.float32),
    scratch_types=[
        pltpu.VMEM((_IPW,), jnp.int32),
        pltpu.VMEM((_HIST, _D), jnp.float32),
        pltpu.VMEM((_HIST, _D), jnp.float32),
        pltpu.SemaphoreType.DMA,
        pltpu.SemaphoreType.DMA,
        pltpu.SemaphoreType.DMA,
        pltpu.SemaphoreType.DMA,
    ],
    compiler_params=pltpu.CompilerParams(use_tc_tiling_on_sc=False),
)
def _gather(idx_hbm, table_hbm, out_hbm, idx_v, rows0, rows1, g0, g1, s0, s1):
    wid = lax.axis_index("s") * _NC + lax.axis_index("c")
    b0 = wid * _BPW

    pltpu.sync_copy(idx_hbm.at[pl.ds(wid * _IPW, _IPW)], idx_v)

    def fire_gather(j, rows, gsem):
        pltpu.async_copy(
            table_hbm.at[idx_v.at[pl.ds(j * _HIST, _C0)]],
            rows.at[pl.ds(0, _C0)],
            gsem,
        )
        pltpu.async_copy(
            table_hbm.at[idx_v.at[pl.ds(j * _HIST + _C0, _C1)]],
            rows.at[pl.ds(_C0, _C1)],
            gsem,
        )

    def drain_gather(rows, gsem):
        pltpu.make_async_copy(table_hbm.at[idx_v.at[pl.ds(0, _C0)]],
                              rows.at[pl.ds(0, _C0)], gsem).wait()
        pltpu.make_async_copy(table_hbm.at[idx_v.at[pl.ds(0, _C1)]],
                              rows.at[pl.ds(_C0, _C1)], gsem).wait()

    def fire_store(j, rows, ssem):
        pltpu.async_copy(rows, out_hbm.at[b0 + j, :, pl.ds(0, _D)], ssem)

    def wait_store(j, rows, ssem):
        pltpu.make_async_copy(rows, out_hbm.at[b0 + j, :, pl.ds(0, _D)],
                              ssem).wait()

    fire_gather(0, rows0, g0)

    def body(jj, carry):
        j = jj * 2
        drain_gather(rows0, g0)

        @pl.when(j > 0)
        def _():
            wait_store(j - 1, rows1, s1)

        fire_gather(j + 1, rows1, g1)
        fire_store(j, rows0, s0)

        drain_gather(rows1, g1)
        wait_store(j, rows0, s0)

        @pl.when(j + 2 < _BPW)
        def _():
            fire_gather(j + 2, rows0, g0)

        fire_store(j + 1, rows1, s1)
        return carry

    lax.fori_loop(0, _BPW // 2, body, 0)
    wait_store(_BPW - 1, rows1, s1)


def kernel(traj, table):
    idx = traj.reshape(-1).astype(jnp.int32) * 2
    table_rows = _tpose(table.T).reshape(2 * _V, _D)
    out_p = _gather(idx, table_rows)
    return out_p[:, :, :_D]


# speedup vs baseline: 2.6783x; 1.0289x over previous
import functools

import jax
import jax.numpy as jnp
from jax import lax
from jax.experimental import pallas as pl
from jax.experimental.pallas import tpu as pltpu
from jax.experimental.pallas import tpu_sc as plsc

_V = 1000000
_D = 64
_DP = 128
_BATCH = 4096
_HIST = 200
_B = _BATCH * _HIST
_NC = 2
_NS = 16
_NW = _NC * _NS
_BPW = _BATCH // _NW
_IPW = _BPW * _HIST
_C0 = 128
_C1 = _HIST - _C0
_TBLK = 16384


def _tpose_body(t_ref, o_ref):
    x = jnp.concatenate(
        [t_ref[...], jnp.zeros((_DP - _D, _TBLK), jnp.float32)], axis=0)
    o_ref[...] = x.T


_tpose = pl.pallas_call(
    _tpose_body,
    grid=(pl.cdiv(_V, _TBLK),),
    in_specs=[pl.BlockSpec((_D, _TBLK), lambda i: (0, i))],
    out_specs=pl.BlockSpec((_TBLK, _DP), lambda i: (i, 0)),
    out_shape=jax.ShapeDtypeStruct((_V, _DP), jnp.float32),
)

_mesh = plsc.VectorSubcoreMesh(core_axis_name="c", subcore_axis_name="s")


@functools.partial(
    pl.kernel,
    mesh=_mesh,
    out_type=jax.ShapeDtypeStruct((_BATCH, _HIST, _DP), jnp.float32),
    scratch_types=[
        pltpu.VMEM((_IPW,), jnp.int32),
        pltpu.VMEM((_HIST, _D), jnp.float32),
        pltpu.VMEM((_HIST, _D), jnp.float32),
        pltpu.SemaphoreType.DMA,
        pltpu.SemaphoreType.DMA,
        pltpu.SemaphoreType.DMA,
        pltpu.SemaphoreType.DMA,
    ],
    compiler_params=pltpu.CompilerParams(use_tc_tiling_on_sc=False),
)
def _gather(idx_hbm, table_hbm, out_hbm, idx_v, rows0, rows1, g0, g1, s0, s1):
    wid = lax.axis_index("s") * _NC + lax.axis_index("c")
    b0 = wid * _BPW

    pltpu.sync_copy(idx_hbm.at[pl.ds(wid * _IPW, _IPW)], idx_v)

    def fire_gather(j, rows, gsem):
        pltpu.async_copy(
            table_hbm.at[idx_v.at[pl.ds(j * _HIST, _C0)]],
            rows.at[pl.ds(0, _C0)],
            gsem,
        )
        pltpu.async_copy(
            table_hbm.at[idx_v.at[pl.ds(j * _HIST + _C0, _C1)]],
            rows.at[pl.ds(_C0, _C1)],
            gsem,
        )

    def drain_gather(rows, gsem):
        pltpu.make_async_copy(table_hbm.at[idx_v.at[pl.ds(0, _C0)]],
                              rows.at[pl.ds(0, _C0)], gsem).wait()
        pltpu.make_async_copy(table_hbm.at[idx_v.at[pl.ds(0, _C1)]],
                              rows.at[pl.ds(_C0, _C1)], gsem).wait()

    def fire_store(j, rows, ssem):
        pltpu.async_copy(rows, out_hbm.at[b0 + j, :, pl.ds(0, _D)], ssem)

    def wait_store(j, rows, ssem):
        pltpu.make_async_copy(rows, out_hbm.at[b0 + j, :, pl.ds(0, _D)],
                              ssem).wait()

    fire_gather(0, rows0, g0)

    def body(jj, carry):
        j = jj * 2
        drain_gather(rows0, g0)

        @pl.when(j > 0)
        def _():
            wait_store(j - 1, rows1, s1)

        fire_gather(j + 1, rows1, g1)
        fire_store(j, rows0, s0)

        drain_gather(rows1, g1)
        wait_store(j, rows0, s0)

        @pl.when(j + 2 < _BPW)
        def _():
            fire_gather(j + 2, rows0, g0)

        fire_store(j + 1, rows1, s1)
        return carry

    lax.fori_loop(0, _BPW // 2, body, 0)
    wait_store(_BPW - 1, rows1, s1)


def kernel(traj, table):
    idx = traj.reshape(-1).astype(jnp.int32) * 2
    table_rows = _tpose(table.T).reshape(2 * _V, _D)
    out_p = _gather(idx, table_rows)
    return out_p[:, :, :_D]
